# Initial kernel scaffold; baseline (speedup 1.0000x reference)
import jax
import jax.numpy as jnp
from jax.experimental import pallas as pl


def kernel(q_feats, s_feats, q_points, s_points, downsample_indices, W1, b1, W2, b2, W3, b3):
    raise NotImplementedError("write your pallas kernel here")


# speedup vs baseline: 20.2933x; 20.2933x over previous
import jax
import jax.numpy as jnp
import numpy as np
from jax import lax
from jax.experimental import pallas as pl
from jax.experimental.pallas import tpu as pltpu
from jax.experimental.pallas import tpu_sc as plsc

_B, _N, _M, _K, _C = 4, 8192, 32768, 16, 64
_P = _B * _N

_R1 = 1024

def _weights_body(s_ref, q_ref, idx_ref, w_ref, gidx_ref):
    i = pl.program_id(0)
    s = s_ref[...]
    q = q_ref[...]
    r0 = lax.broadcasted_iota(jnp.int32, (3, _K * 3), 0)
    c0 = lax.broadcasted_iota(jnp.int32, (3, _K * 3), 1)
    E = (c0 % 3 == r0).astype(jnp.float32)
    r1 = lax.broadcasted_iota(jnp.int32, (_K * 3, _K), 0)
    c1 = lax.broadcasted_iota(jnp.int32, (_K * 3, _K), 1)
    SEL = (r1 // 3 == c1).astype(jnp.float32)
    qt = jnp.dot(q, E, preferred_element_type=jnp.float32)
    d = s - qt
    d2 = jnp.dot(d * d, SEL, preferred_element_type=jnp.float32)
    w = 1.0 / (d2 + 1e-8)
    w_ref[...] = w / jnp.sum(w, axis=1, keepdims=True)
    b = i // (_N // _R1)
    gidx_ref[...] = idx_ref[...] + b * _M


def _compute_weights(s2, q2, idx2, interpret=False):
    return pl.pallas_call(
        _weights_body,
        grid=(_P // _R1,),
        in_specs=[
            pl.BlockSpec((_R1, _K * 3), lambda i: (i, 0)),
            pl.BlockSpec((_R1, 3), lambda i: (i, 0)),
            pl.BlockSpec((_R1, _K), lambda i: (i, 0)),
        ],
        out_specs=[
            pl.BlockSpec((_R1, _K), lambda i: (i, 0)),
            pl.BlockSpec((_R1, _K), lambda i: (i, 0)),
        ],
        out_shape=[
            jax.ShapeDtypeStruct((_P, _K), jnp.float32),
            jax.ShapeDtypeStruct((_P, _K), jnp.int32),
        ],
        interpret=interpret,
    )(s2, q2, idx2)


_NSC, _NSUB, _L = 2, 16, 16
_NW = _NSC * _NSUB
_PPW = _P // _NW
_GP = 16
_NCH = _GP * _K // 128
_NG = _PPW // _GP
_CPW = _PPW * _K // 128


def _sc_pool_body(table, idx2, w2, out,
                  idx_v, w_v, rows_v, out_v,
                  gsem0, gsem1, xsem0, xsem1, osem0, osem1):
    wid = lax.axis_index("s") * _NSC + lax.axis_index("c")
    crow0 = wid * _CPW
    prow0 = wid * _PPW
    gsems = (gsem0, gsem1)
    xsems = (xsem0, xsem1)
    osems = (osem0, osem1)

    def stage_idx(g, b):
        r = crow0 + g * _NCH
        pltpu.async_copy(idx2.at[pl.ds(r, _NCH)], idx_v.at[b], xsems[b])
        pltpu.async_copy(w2.at[pl.ds(r, _NCH)], w_v.at[b], xsems[b])

    def wait_idx(g, b):
        r = crow0 + g * _NCH
        pltpu.make_async_copy(idx2.at[pl.ds(r, _NCH)], idx_v.at[b], xsems[b]).wait()
        pltpu.make_async_copy(w2.at[pl.ds(r, _NCH)], w_v.at[b], xsems[b]).wait()

    def fire_gathers(b):
        for c in range(_NCH):
            pltpu.async_copy(table.at[idx_v.at[b, c]], rows_v.at[b, c], gsems[b])

    def wait_gathers(b):
        for c in range(_NCH):
            pltpu.make_async_copy(
                table.at[idx_v.at[b, c]], rows_v.at[b, c], gsems[b]).wait()

    def wait_out(g, b):
        pltpu.make_async_copy(
            out_v.at[b], out.at[pl.ds(prow0 + g * _GP, _GP)], osems[b]).wait()

    def compute(g, b):
        for p in range(_GP):
            c, pp = divmod(p, 8)
            base = pp * _K
            acc = [None] * 4
            w16 = w_v[b, c, base:base + _K]
            for k in range(_K):
                wk = w16[k]
                for j in range(4):
                    r = rows_v[b, c, base + k, j * _L:(j + 1) * _L]
                    acc[j] = wk * r if k == 0 else acc[j] + wk * r
            for j in range(4):
                out_v[b, p, j * _L:(j + 1) * _L] = acc[j]
        pltpu.async_copy(out_v.at[b], out.at[pl.ds(prow0 + g * _GP, _GP)],
                         osems[b])

    stage_idx(0, 0)
    wait_idx(0, 0)
    fire_gathers(0)
    stage_idx(1, 1)

    def body(gg, carry):
        for b in (0, 1):
            g = 2 * gg + b
            wait_gathers(b)
            if b == 0:
                wait_idx(g + 1, 1)
                fire_gathers(1)
            else:
                @pl.when(gg < (_NG // 2 - 1))
                def _():
                    wait_idx(g + 1, 0)
                    fire_gathers(0)

            @pl.when(gg >= 1)
            def _():
                wait_out(g - 2, b)

            compute(g, b)

            @pl.when(gg < (_NG // 2 - 1))
            def _():
                stage_idx(g + 2, b)
        return carry

    lax.fori_loop(0, _NG // 2, body, 0)
    wait_out(_NG - 2, 0)
    wait_out(_NG - 1, 1)


def _sc_pool(table, gidx, w):
    kern = pl.kernel(
        _sc_pool_body,
        out_type=jax.ShapeDtypeStruct((_P, _C), jnp.float32),
        mesh=plsc.VectorSubcoreMesh(
            core_axis_name="c", subcore_axis_name="s",
            num_cores=_NSC, num_subcores=_NSUB),
        compiler_params=pltpu.CompilerParams(use_tc_tiling_on_sc=False),
        scratch_types=[
            pltpu.VMEM((2, _NCH, 128), jnp.int32),
            pltpu.VMEM((2, _NCH, 128), jnp.float32),
            pltpu.VMEM((2, _NCH, 128, _C), jnp.float32),
            pltpu.VMEM((2, _GP, _C), jnp.float32),
            pltpu.SemaphoreType.DMA,
            pltpu.SemaphoreType.DMA,
            pltpu.SemaphoreType.DMA,
            pltpu.SemaphoreType.DMA,
            pltpu.SemaphoreType.DMA,
            pltpu.SemaphoreType.DMA,
        ],
    )
    return kern(table, gidx, w)


_R2 = 2048


def _lrelu(v):
    return jnp.where(v >= 0, v, 0.1 * v)


def _mlp_body(x_ref, qf_ref, w1_ref, b1_ref, w2_ref, b2_ref, w3_ref, b3_ref,
              o_ref):
    x = x_ref[...]
    h = _lrelu(jnp.dot(x, w1_ref[...], preferred_element_type=jnp.float32)
               + b1_ref[...])
    h = _lrelu(jnp.dot(h, w2_ref[...], preferred_element_type=jnp.float32)
               + b2_ref[...])
    o_ref[...] = _lrelu(
        jnp.dot(h + qf_ref[...], w3_ref[...],
                preferred_element_type=jnp.float32) + b3_ref[...])


def _mlp(latent, qf, W1, b1, W2, b2, W3, b3, interpret=False):
    row_spec = pl.BlockSpec((_R2, _C), lambda i: (i, 0))
    w_spec = pl.BlockSpec((_C, _C), lambda i: (0, 0))
    b_spec = pl.BlockSpec((1, _C), lambda i: (0, 0))
    return pl.pallas_call(
        _mlp_body,
        grid=(_P // _R2,),
        in_specs=[row_spec, row_spec, w_spec, b_spec, w_spec, b_spec,
                  w_spec, b_spec],
        out_specs=row_spec,
        out_shape=jax.ShapeDtypeStruct((_P, _C), jnp.float32),
        interpret=interpret,
    )(latent, qf, W1, b1, W2, b2, W3, b3)


def kernel(q_feats, s_feats, q_points, s_points, downsample_indices,
           W1, b1, W2, b2, W3, b3):
    s2 = s_points.reshape(_P, _K * 3)
    q2 = q_points.reshape(_P, 3)
    idx = downsample_indices.reshape(_P, _K)
    w, gidx = _compute_weights(s2, q2, idx)
    table = s_feats.reshape(_B * _M, _C)
    latent = _sc_pool(table,
                      gidx.reshape(_P * _K // 128, 128),
                      w.reshape(_P * _K // 128, 128))
    out = _mlp(latent, q_feats.reshape(_P, _C),
               W1, b1.reshape(1, _C), W2, b2.reshape(1, _C),
               W3, b3.reshape(1, _C))
    return out.reshape(_B, _N, _C)
